# baseline (speedup 1.0000x reference)
import functools

import jax
import jax.numpy as jnp
from jax import lax
from jax.experimental import pallas as pl
from jax.experimental.pallas import tpu as pltpu
from jax.experimental.pallas import tpu_sc as plsc

B = 4096
IN = 512
H1 = 512
H2 = 256
C = 16

NC = 2
NS = 16
L = 16
NW = NC * NS
ROWS_PER_W = B // NW
PASS_B = 64

_SPLAT_DNUMS = lax.GatherDimensionNumbers(
    offset_dims=(), collapsed_slice_dims=(0,), start_index_map=(0,))


def _splat_lane(vec, k):
    idx = jnp.full((L, 1), k, jnp.int32)
    return lax.gather(vec, idx, _SPLAT_DNUMS, slice_sizes=(1,),
                      mode=lax.GatherScatterMode.PROMISE_IN_BOUNDS)


def _layer_loop(in_buf, conn_buf, op_buf, out_buf, out_f, iota):
    def body(j, carry):
        crow = conn_buf[j, :]
        opv = op_buf[j, :]
        jsp = jnp.full((L,), 0, jnp.int32) + j
        for bc in range(PASS_B // L):
            biota = iota + (bc * L)
            mn = jnp.full((L,), jnp.inf, jnp.float32)
            mx = jnp.full((L,), -jnp.inf, jnp.float32)
            for k in range(C):
                csp = _splat_lane(crow, k)
                v = plsc.load_gather(in_buf, [biota, csp])
                mn = jnp.minimum(mn, v)
                mx = jnp.maximum(mx, v)
            res = jnp.where(opv == 1, mx, mn)
            plsc.store_scatter(out_buf, [biota, jsp], res)
        return carry

    lax.fori_loop(0, out_f, body, 0)


def _make_kernel():
    mesh = plsc.VectorSubcoreMesh(core_axis_name="c", subcore_axis_name="s")

    @functools.partial(
        pl.kernel,
        mesh=mesh,
        out_type=jax.ShapeDtypeStruct((B, IN), jnp.float32),
        compiler_params=pltpu.CompilerParams(
            use_tc_tiling_on_sc=False, needs_layout_passes=False),
        scratch_types=[
            pltpu.VMEM((PASS_B, IN), jnp.float32),
            pltpu.VMEM((PASS_B, H1), jnp.float32),
            pltpu.VMEM((PASS_B, H2), jnp.float32),
            pltpu.VMEM((H1, C), jnp.int32),
            pltpu.VMEM((H2, C), jnp.int32),
            pltpu.VMEM((IN, C), jnp.int32),
            pltpu.VMEM((H1, C), jnp.int32),
            pltpu.VMEM((H2, C), jnp.int32),
            pltpu.VMEM((IN, C), jnp.int32),
        ],
    )
    def k(x_hbm, c1_hbm, c2_hbm, co_hbm, o1_hbm, o2_hbm, oo_hbm, out_hbm,
          xbuf, h1buf, h2buf, c1b, c2b, cob, o1b, o2b, oob):
        wid = lax.axis_index("s") * NC + lax.axis_index("c")
        pltpu.sync_copy(c1_hbm, c1b)
        pltpu.sync_copy(c2_hbm, c2b)
        pltpu.sync_copy(co_hbm, cob)
        pltpu.sync_copy(o1_hbm, o1b)
        pltpu.sync_copy(o2_hbm, o2b)
        pltpu.sync_copy(oo_hbm, oob)
        iota = lax.iota(jnp.int32, L)
        for p in range(ROWS_PER_W // PASS_B):
            base = wid * ROWS_PER_W + p * PASS_B
            pltpu.sync_copy(x_hbm.at[pl.ds(base, PASS_B)], xbuf)
            _layer_loop(xbuf, c1b, o1b, h1buf, H1, iota)
            _layer_loop(h1buf, c2b, o2b, h2buf, H2, iota)
            _layer_loop(h2buf, cob, oob, xbuf, IN, iota)
            pltpu.sync_copy(xbuf, out_hbm.at[pl.ds(base, PASS_B)])

    return k


@jax.jit
def kernel(x, conn1, conn2, conn_out, op1, op2, op_out):
    o1 = jnp.broadcast_to(op1[:, None], (H1, C))
    o2 = jnp.broadcast_to(op2[:, None], (H2, C))
    oo = jnp.broadcast_to(op_out[:, None], (IN, C))
    return _make_kernel()(x, conn1, conn2, conn_out, o1, o2, oo)


# speedup vs baseline: 5.1778x; 5.1778x over previous
import functools

import jax
import jax.numpy as jnp
from jax import lax
from jax.experimental import pallas as pl
from jax.experimental.pallas import tpu as pltpu
from jax.experimental.pallas import tpu_sc as plsc

B = 4096
IN = 512
H1 = 512
H2 = 256
C = 16

NC = 2
NS = 16
L = 16
NW = NC * NS
ROWS_PER_W = B // NW
PASS_B = 64
NCHUNK = B // PASS_B

_SPLAT_DNUMS = lax.GatherDimensionNumbers(
    offset_dims=(), collapsed_slice_dims=(0,), start_index_map=(0,))


def _splat_lane(vec, k):
    idx = jnp.full((L, 1), k, jnp.int32)
    return lax.gather(vec, idx, _SPLAT_DNUMS, slice_sizes=(1,),
                      mode=lax.GatherScatterMode.PROMISE_IN_BOUNDS)


def _layer_loop(in_buf, conn_buf, op_buf, out_buf, out_f, iota):
    def body(j, carry):
        crow = conn_buf[j, :]
        opv = op_buf[j, :]
        jsp = jnp.full((L,), 0, jnp.int32) + j
        for bc in range(PASS_B // L):
            biota = iota + (bc * L)
            mn = jnp.full((L,), jnp.inf, jnp.float32)
            mx = jnp.full((L,), -jnp.inf, jnp.float32)
            for k in range(C):
                csp = _splat_lane(crow, k)
                v = plsc.load_gather(in_buf, [csp, biota])
                mn = jnp.minimum(mn, v)
                mx = jnp.maximum(mx, v)
            res = jnp.where(opv == 1, mx, mn)
            plsc.store_scatter(out_buf, [jsp, biota], res)
        return carry

    lax.fori_loop(0, out_f, body, 0)


def _make_kernel():
    mesh = plsc.VectorSubcoreMesh(core_axis_name="c", subcore_axis_name="s")

    @functools.partial(
        pl.kernel,
        mesh=mesh,
        out_type=jax.ShapeDtypeStruct((NCHUNK, IN, PASS_B), jnp.float32),
        compiler_params=pltpu.CompilerParams(
            use_tc_tiling_on_sc=False, needs_layout_passes=False),
        scratch_types=[
            pltpu.VMEM((IN, PASS_B), jnp.float32),
            pltpu.VMEM((H1, PASS_B), jnp.float32),
            pltpu.VMEM((H2, PASS_B), jnp.float32),
            pltpu.VMEM((H1, C), jnp.int32),
            pltpu.VMEM((H2, C), jnp.int32),
            pltpu.VMEM((IN, C), jnp.int32),
            pltpu.VMEM((H1, C), jnp.int32),
            pltpu.VMEM((H2, C), jnp.int32),
            pltpu.VMEM((IN, C), jnp.int32),
        ],
    )
    def k(xt_hbm, c1_hbm, c2_hbm, co_hbm, o1_hbm, o2_hbm, oo_hbm, out_hbm,
          xbuf, h1buf, h2buf, c1b, c2b, cob, o1b, o2b, oob):
        wid = lax.axis_index("s") * NC + lax.axis_index("c")
        pltpu.sync_copy(c1_hbm, c1b)
        pltpu.sync_copy(c2_hbm, c2b)
        pltpu.sync_copy(co_hbm, cob)
        pltpu.sync_copy(o1_hbm, o1b)
        pltpu.sync_copy(o2_hbm, o2b)
        pltpu.sync_copy(oo_hbm, oob)
        iota = lax.iota(jnp.int32, L)
        for p in range(ROWS_PER_W // PASS_B):
            chunk = wid * (ROWS_PER_W // PASS_B) + p
            pltpu.sync_copy(xt_hbm.at[chunk], xbuf)
            _layer_loop(xbuf, c1b, o1b, h1buf, H1, iota)
            _layer_loop(h1buf, c2b, o2b, h2buf, H2, iota)
            _layer_loop(h2buf, cob, oob, xbuf, IN, iota)
            pltpu.sync_copy(xbuf, out_hbm.at[chunk])

    return k


@jax.jit
def kernel(x, conn1, conn2, conn_out, op1, op2, op_out):
    o1 = jnp.broadcast_to(op1[:, None], (H1, C))
    o2 = jnp.broadcast_to(op2[:, None], (H2, C))
    oo = jnp.broadcast_to(op_out[:, None], (IN, C))
    xt = x.reshape(NCHUNK, PASS_B, IN).transpose(0, 2, 1)
    outt = _make_kernel()(xt, conn1, conn2, conn_out, o1, o2, oo)
    return outt.transpose(0, 2, 1).reshape(B, IN)


# speedup vs baseline: 5.2543x; 1.0148x over previous
import functools

import jax
import jax.numpy as jnp
from jax import lax
from jax.experimental import pallas as pl
from jax.experimental.pallas import tpu as pltpu
from jax.experimental.pallas import tpu_sc as plsc

B = 4096
IN = 512
H1 = 512
H2 = 256
C = 16

NC = 2
NS = 16
L = 16
NW = NC * NS
ROWS_PER_W = B // NW
PASS_B = 64
NCHUNK = B // PASS_B

_SPLAT_DNUMS = lax.GatherDimensionNumbers(
    offset_dims=(), collapsed_slice_dims=(0,), start_index_map=(0,))


def _splat_lane(vec, k):
    idx = jnp.full((L, 1), k, jnp.int32)
    return lax.gather(vec, idx, _SPLAT_DNUMS, slice_sizes=(1,),
                      mode=lax.GatherScatterMode.PROMISE_IN_BOUNDS)


def _layer_loop(in_buf, in_words, conn_buf, op_buf, out_buf, out_f, iota):
    def body(j, carry):
        crow = conn_buf[j, :]
        opv = op_buf[j, :]
        jsp = jnp.full((L,), 0, jnp.int32) + j * PASS_B
        for bc in range(PASS_B // L):
            base = bc * L
            view = in_buf.at[pl.ds(base, in_words - base)]
            csp = _splat_lane(crow, 0)
            mn = plsc.load_gather(view, [csp])
            mx = mn
            for k in range(1, C):
                csp = _splat_lane(crow, k)
                v = plsc.load_gather(view, [csp])
                mn = jnp.minimum(mn, v)
                mx = jnp.maximum(mx, v)
            res = jnp.where(opv == 1, mx, mn)
            plsc.store_scatter(out_buf, [jsp + (iota + base)], res)
        return carry

    lax.fori_loop(0, out_f, body, 0)


def _make_kernel():
    mesh = plsc.VectorSubcoreMesh(core_axis_name="c", subcore_axis_name="s")

    @functools.partial(
        pl.kernel,
        mesh=mesh,
        out_type=jax.ShapeDtypeStruct((NCHUNK, IN * PASS_B), jnp.float32),
        compiler_params=pltpu.CompilerParams(
            use_tc_tiling_on_sc=False, needs_layout_passes=False),
        scratch_types=[
            pltpu.VMEM((IN * PASS_B,), jnp.float32),
            pltpu.VMEM((H1 * PASS_B,), jnp.float32),
            pltpu.VMEM((H2 * PASS_B,), jnp.float32),
            pltpu.VMEM((H1, C), jnp.int32),
            pltpu.VMEM((H2, C), jnp.int32),
            pltpu.VMEM((IN, C), jnp.int32),
            pltpu.VMEM((H1, C), jnp.int32),
            pltpu.VMEM((H2, C), jnp.int32),
            pltpu.VMEM((IN, C), jnp.int32),
        ],
    )
    def k(xt_hbm, c1_hbm, c2_hbm, co_hbm, o1_hbm, o2_hbm, oo_hbm, out_hbm,
          xbuf, h1buf, h2buf, c1b, c2b, cob, o1b, o2b, oob):
        wid = lax.axis_index("s") * NC + lax.axis_index("c")
        pltpu.sync_copy(c1_hbm, c1b)
        pltpu.sync_copy(c2_hbm, c2b)
        pltpu.sync_copy(co_hbm, cob)
        pltpu.sync_copy(o1_hbm, o1b)
        pltpu.sync_copy(o2_hbm, o2b)
        pltpu.sync_copy(oo_hbm, oob)
        iota = lax.iota(jnp.int32, L)
        for p in range(ROWS_PER_W // PASS_B):
            chunk = wid * (ROWS_PER_W // PASS_B) + p
            pltpu.sync_copy(xt_hbm.at[chunk], xbuf)
            _layer_loop(xbuf, IN * PASS_B, c1b, o1b, h1buf, H1, iota)
            _layer_loop(h1buf, H1 * PASS_B, c2b, o2b, h2buf, H2, iota)
            _layer_loop(h2buf, H2 * PASS_B, cob, oob, xbuf, IN, iota)
            pltpu.sync_copy(xbuf, out_hbm.at[chunk])

    return k


@jax.jit
def kernel(x, conn1, conn2, conn_out, op1, op2, op_out):
    o1 = jnp.broadcast_to(op1[:, None], (H1, C))
    o2 = jnp.broadcast_to(op2[:, None], (H2, C))
    oo = jnp.broadcast_to(op_out[:, None], (IN, C))
    xt = x.reshape(NCHUNK, PASS_B, IN).transpose(0, 2, 1).reshape(
        NCHUNK, IN * PASS_B)
    outt = _make_kernel()(xt, conn1 * PASS_B, conn2 * PASS_B,
                          conn_out * PASS_B, o1, o2, oo)
    return outt.reshape(NCHUNK, IN, PASS_B).transpose(0, 2, 1).reshape(B, IN)
